# Initial kernel scaffold; baseline (speedup 1.0000x reference)
import jax
import jax.numpy as jnp
from jax.experimental import pallas as pl


def kernel(feat, edge_index, efeat, W_edge, b_edge, bias):
    raise NotImplementedError("write your pallas kernel here")


# speedup vs baseline: 2.1281x; 2.1281x over previous
import functools

import jax
import jax.numpy as jnp
from jax import lax
from jax.experimental import pallas as pl
from jax.experimental.pallas import tpu as pltpu
from jax.experimental.pallas import tpu_sc as plsc

N = 10000
E = 160000
F_IN = 32
F_OUT = 32
D_EDGE = 16

NC = 2
NS = 16
NW = NC * NS
PER_W = E // NW
C = 125
NCH = PER_W // C
GF = 8
NG = NCH // GF
NP = 10240
SLAB = NP // NS

ROW = F_OUT + 16

TE = 2000
TN = 2048


def _gather_body(feat_hbm, src_hbm, xsrc_hbm, idx_v, rows_v, sem):
    c = lax.axis_index("c")
    s = lax.axis_index("s")
    wid = c * NS + s
    pltpu.sync_copy(src_hbm.at[pl.ds(wid * NCH, NCH)], idx_v)

    def group(g, carry):
        cps = [
            pltpu.async_copy(
                feat_hbm.at[idx_v.at[g * GF + b]],
                rows_v.at[pl.ds(b * C, C)],
                sem,
            )
            for b in range(GF)
        ]
        for cp in cps:
            cp.wait()
        pltpu.sync_copy(
            rows_v, xsrc_hbm.at[pl.ds(wid * PER_W + g * (GF * C), GF * C)]
        )
        return carry

    lax.fori_loop(0, NG, group, 0)


def _sc_gather(feat, src2d):
    mesh = plsc.VectorSubcoreMesh(core_axis_name="c", subcore_axis_name="s")
    return pl.kernel(
        _gather_body,
        out_type=jax.ShapeDtypeStruct((E, F_IN), jnp.float32),
        mesh=mesh,
        compiler_params=pltpu.CompilerParams(use_tc_tiling_on_sc=False),
        scratch_types=[
            pltpu.VMEM((NCH, C), jnp.int32),
            pltpu.VMEM((GF * C, F_IN), jnp.float32),
            pltpu.SemaphoreType.DMA,
        ],
    )(feat, src2d)


def _msg_body(x_ref, ef_ref, w_ref, o_ref):
    z = jnp.dot(x_ref[...], w_ref[...], preferred_element_type=jnp.float32)
    ef = ef_ref[...]
    acc = z[:, 512:544]
    for d in range(D_EDGE):
        acc = acc + ef[:, d : d + 1] * z[:, 32 * d : 32 * (d + 1)]
    ones_blk = (
        lax.broadcasted_iota(jnp.int32, (TE, 16), 1) == 0
    ).astype(jnp.float32)
    o_ref[...] = jnp.concatenate([acc, ones_blk], axis=1)


def _tc_msg(xsrc, efeat, wfull):
    return pl.pallas_call(
        _msg_body,
        grid=(E // TE,),
        in_specs=[
            pl.BlockSpec((TE, F_IN), lambda i: (i, 0)),
            pl.BlockSpec((TE, D_EDGE), lambda i: (i, 0)),
            pl.BlockSpec((F_IN, 544), lambda i: (0, 0)),
        ],
        out_specs=pl.BlockSpec((TE, ROW), lambda i: (i, 0)),
        out_shape=jax.ShapeDtypeStruct((E, ROW), jnp.float32),
    )(xsrc, efeat, wfull)


def _scatter_body(msg_hbm, dst_hbm, zeros_hbm, acc_hbm, idx_v, rows_v, slab_v, sem, shared):
    c = lax.axis_index("c")
    s = lax.axis_index("s")
    wid = c * NS + s

    if True:
        pltpu.sync_copy(zeros_hbm.at[pl.ds(s * SLAB, SLAB)], slab_v)
        pltpu.sync_copy(slab_v, shared.at[pl.ds(s * SLAB, SLAB)])
        plsc.subcore_barrier()

        pltpu.sync_copy(dst_hbm.at[pl.ds(wid * NCH, NCH)], idx_v)

        def group(g, carry):
            pltpu.sync_copy(
                msg_hbm.at[pl.ds(wid * PER_W + g * (GF * C), GF * C)], rows_v
            )
            for b in range(GF):
                pltpu.sync_copy(
                    rows_v.at[pl.ds(b * C, C)],
                    shared.at[idx_v.at[g * GF + b]],
                    add=True,
                )
            return carry

        lax.fori_loop(0, NG, group, 0)
        plsc.subcore_barrier()

        pltpu.sync_copy(shared.at[pl.ds(s * SLAB, SLAB)], slab_v)
        pltpu.sync_copy(slab_v, acc_hbm.at[c, pl.ds(s * SLAB, SLAB)])


def _sc_scatter(msg48, dst2d, zeros):
    mesh = plsc.VectorSubcoreMesh(core_axis_name="c", subcore_axis_name="s")
    return pl.kernel(
        _scatter_body,
        out_type=jax.ShapeDtypeStruct((NC, NP, ROW), jnp.float32),
        mesh=mesh,
        compiler_params=pltpu.CompilerParams(use_tc_tiling_on_sc=False),
        scratch_types=[
            pltpu.VMEM((NCH, C), jnp.int32),
            pltpu.VMEM((GF * C, ROW), jnp.float32),
            pltpu.VMEM((SLAB, ROW), jnp.float32),
            pltpu.SemaphoreType.DMA,
            pltpu.VMEM_SHARED((NP, ROW), jnp.float32),
        ],
    )(msg48, dst2d, zeros)


def _fin_body(a_ref, b_ref, o_ref):
    a = a_ref[...]
    ssum = a[0] + a[1]
    cnt = ssum[:, F_OUT : F_OUT + 1]
    o_ref[...] = ssum[:, :F_OUT] / jnp.maximum(cnt, 1.0) + b_ref[...]


def _tc_fin(acc, bias2d):
    return pl.pallas_call(
        _fin_body,
        grid=(NP // TN,),
        in_specs=[
            pl.BlockSpec((NC, TN, ROW), lambda j: (0, j, 0)),
            pl.BlockSpec((1, F_OUT), lambda j: (0, 0)),
        ],
        out_specs=pl.BlockSpec((TN, F_OUT), lambda j: (j, 0)),
        out_shape=jax.ShapeDtypeStruct((N, F_OUT), jnp.float32),
    )(acc, bias2d)


def kernel(feat, edge_index, efeat, W_edge, b_edge, bias):
    src2d = edge_index[0].reshape(E // C, C)
    dst2d = edge_index[1].reshape(E // C, C)
    wcat = W_edge.reshape(D_EDGE, F_IN, F_OUT).transpose(1, 0, 2).reshape(F_IN, 512)
    wfull = jnp.concatenate([wcat, b_edge.reshape(F_IN, F_OUT)], axis=1)

    xsrc = _sc_gather(feat, src2d)
    msg48 = _tc_msg(xsrc, efeat, wfull)
    acc = _sc_scatter(msg48, dst2d, jnp.zeros((NP, ROW), jnp.float32))
    return _tc_fin(acc, bias.reshape(1, F_OUT))


# speedup vs baseline: 4.3654x; 2.0512x over previous
import functools

import jax
import jax.numpy as jnp
from jax import lax
from jax.experimental import pallas as pl
from jax.experimental.pallas import tpu as pltpu
from jax.experimental.pallas import tpu_sc as plsc

N = 10000
E = 160000
F_IN = 32
F_OUT = 32
D_EDGE = 16

NC = 2
NS = 16
NW = NC * NS
PER_W = E // NW
C = 125
NCH = PER_W // C
GF = 8
NG = NCH // GF
NP = 10240
SLAB = NP // NS

ROW = F_OUT + 16

TE = 2000
TN = 2048


def _gather_body(feat_hbm, src_hbm, xsrc_hbm, idx_v, rows_v, sem):
    c = lax.axis_index("c")
    s = lax.axis_index("s")
    wid = c * NS + s
    pltpu.sync_copy(src_hbm.at[pl.ds(wid * NCH, NCH)], idx_v)

    def group(g, carry):
        cps = [
            pltpu.async_copy(
                feat_hbm.at[idx_v.at[g * GF + b]],
                rows_v.at[pl.ds(b * C, C)],
                sem,
            )
            for b in range(GF)
        ]
        for cp in cps:
            cp.wait()
        pltpu.sync_copy(
            rows_v, xsrc_hbm.at[pl.ds(wid * PER_W + g * (GF * C), GF * C)]
        )
        return carry

    lax.fori_loop(0, NG, group, 0)


def _sc_gather(feat, src2d):
    mesh = plsc.VectorSubcoreMesh(core_axis_name="c", subcore_axis_name="s")
    return pl.kernel(
        _gather_body,
        out_type=jax.ShapeDtypeStruct((E, F_IN), jnp.float32),
        mesh=mesh,
        compiler_params=pltpu.CompilerParams(use_tc_tiling_on_sc=False),
        scratch_types=[
            pltpu.VMEM((NCH, C), jnp.int32),
            pltpu.VMEM((GF * C, F_IN), jnp.float32),
            pltpu.SemaphoreType.DMA,
        ],
    )(feat, src2d)


def _msg_body(x_ref, ef_ref, w_ref, r_ref, o_ref):
    z = jnp.dot(
        x_ref[...].astype(jnp.bfloat16),
        w_ref[...].astype(jnp.bfloat16),
        preferred_element_type=jnp.float32,
    )
    efr = jnp.dot(
        ef_ref[...].astype(jnp.bfloat16),
        r_ref[...].astype(jnp.bfloat16),
        preferred_element_type=jnp.float32,
    )
    mf = efr * z[:, 0:512]
    s1 = mf[:, 0:128] + mf[:, 128:256] + mf[:, 256:384] + mf[:, 384:512]
    acc = (
        z[:, 512:544]
        + s1[:, 0:32]
        + s1[:, 32:64]
        + s1[:, 64:96]
        + s1[:, 96:128]
    )
    ones_blk = (
        lax.broadcasted_iota(jnp.int32, (TE, 16), 1) == 0
    ).astype(jnp.float32)
    o_ref[...] = jnp.concatenate([acc, ones_blk], axis=1)


def _tc_msg(xsrc, efeat, wfull, rexp):
    return pl.pallas_call(
        _msg_body,
        grid=(E // TE,),
        in_specs=[
            pl.BlockSpec((TE, F_IN), lambda i: (i, 0)),
            pl.BlockSpec((TE, D_EDGE), lambda i: (i, 0)),
            pl.BlockSpec((F_IN, 544), lambda i: (0, 0)),
            pl.BlockSpec((D_EDGE, 512), lambda i: (0, 0)),
        ],
        out_specs=pl.BlockSpec((TE, ROW), lambda i: (i, 0)),
        out_shape=jax.ShapeDtypeStruct((E, ROW), jnp.float32),
    )(xsrc, efeat, wfull, rexp)


def _scatter_body(msg_hbm, dst_hbm, zeros_hbm, acc_hbm, idx_v, rows_v, slab_v, sem, shared):
    c = lax.axis_index("c")
    s = lax.axis_index("s")
    wid = c * NS + s

    if True:
        pltpu.sync_copy(zeros_hbm.at[pl.ds(s * SLAB, SLAB)], slab_v)
        pltpu.sync_copy(slab_v, shared.at[pl.ds(s * SLAB, SLAB)])
        plsc.subcore_barrier()

        pltpu.sync_copy(dst_hbm.at[pl.ds(wid * NCH, NCH)], idx_v)

        def group(g, carry):
            pltpu.sync_copy(
                msg_hbm.at[pl.ds(wid * PER_W + g * (GF * C), GF * C)], rows_v
            )
            cps = [
                pltpu.async_copy(
                    rows_v.at[pl.ds(b * C, C)],
                    shared.at[idx_v.at[g * GF + b]],
                    sem,
                    add=True,
                )
                for b in range(GF)
            ]
            for cp in cps:
                cp.wait()
            return carry

        lax.fori_loop(0, NG, group, 0)
        plsc.subcore_barrier()

        pltpu.sync_copy(shared.at[pl.ds(s * SLAB, SLAB)], slab_v)
        pltpu.sync_copy(slab_v, acc_hbm.at[c, pl.ds(s * SLAB, SLAB)])


def _sc_scatter(msg48, dst2d, zeros):
    mesh = plsc.VectorSubcoreMesh(core_axis_name="c", subcore_axis_name="s")
    return pl.kernel(
        _scatter_body,
        out_type=jax.ShapeDtypeStruct((NC, NP, ROW), jnp.float32),
        mesh=mesh,
        compiler_params=pltpu.CompilerParams(use_tc_tiling_on_sc=False),
        scratch_types=[
            pltpu.VMEM((NCH, C), jnp.int32),
            pltpu.VMEM((GF * C, ROW), jnp.float32),
            pltpu.VMEM((SLAB, ROW), jnp.float32),
            pltpu.SemaphoreType.DMA,
            pltpu.VMEM_SHARED((NP, ROW), jnp.float32),
        ],
    )(msg48, dst2d, zeros)


def _fin_body(a_ref, b_ref, o_ref):
    a = a_ref[...]
    ssum = a[0] + a[1]
    cnt = ssum[:, F_OUT : F_OUT + 1]
    o_ref[...] = ssum[:, :F_OUT] / jnp.maximum(cnt, 1.0) + b_ref[...]


def _tc_fin(acc, bias2d):
    return pl.pallas_call(
        _fin_body,
        grid=(NP // TN,),
        in_specs=[
            pl.BlockSpec((NC, TN, ROW), lambda j: (0, j, 0)),
            pl.BlockSpec((1, F_OUT), lambda j: (0, 0)),
        ],
        out_specs=pl.BlockSpec((TN, F_OUT), lambda j: (j, 0)),
        out_shape=jax.ShapeDtypeStruct((N, F_OUT), jnp.float32),
    )(acc, bias2d)


def kernel(feat, edge_index, efeat, W_edge, b_edge, bias):
    src2d = edge_index[0].reshape(E // C, C)
    dst2d = edge_index[1].reshape(E // C, C)
    wcat = W_edge.reshape(D_EDGE, F_IN, F_OUT).transpose(1, 0, 2).reshape(F_IN, 512)
    wfull = jnp.concatenate([wcat, b_edge.reshape(F_IN, F_OUT)], axis=1)

    rexp = (
        jnp.arange(512, dtype=jnp.int32)[None, :] // F_OUT
        == jnp.arange(D_EDGE, dtype=jnp.int32)[:, None]
    ).astype(jnp.float32)
    xsrc = _sc_gather(feat, src2d)
    msg48 = _tc_msg(xsrc, efeat, wfull, rexp)
    acc = _sc_scatter(msg48, dst2d, jnp.zeros((NP, ROW), jnp.float32))
    return _tc_fin(acc, bias.reshape(1, F_OUT))
